# Initial kernel scaffold; baseline (speedup 1.0000x reference)
import jax
import jax.numpy as jnp
from jax.experimental import pallas as pl


def kernel(movie_id, movie_title, movie_genres, movie_id_table, title_table, genre_table, W1, b1, W2, b2):
    raise NotImplementedError("write your pallas kernel here")


# speedup vs baseline: 1.4091x; 1.4091x over previous
import functools

import jax
import jax.numpy as jnp
from jax import lax
from jax.experimental import pallas as pl
from jax.experimental.pallas import tpu as pltpu
from jax.experimental.pallas import tpu_sc as plsc

_B = 4096
_LT = 20
_LG = 8
_DT = 64
_DI = 64
_DG = 32
_VG = 32
_GW = 128
_BB = 512


def _sc_gather(title_table, movie_id_table, tidx2d, iidx2d):
    n_t = tidx2d.shape[1]
    n_i = iidx2d.shape[1]
    mesh = plsc.VectorSubcoreMesh(core_axis_name="c", subcore_axis_name="s")

    @functools.partial(
        pl.kernel,
        out_type=(
            jax.ShapeDtypeStruct((n_t, _DT), jnp.float32),
            jax.ShapeDtypeStruct((n_i, _DI), jnp.float32),
        ),
        mesh=mesh,
        compiler_params=pltpu.CompilerParams(use_tc_tiling_on_sc=False),
    )
    def k(ttab_hbm, itab_hbm, tidx_hbm, iidx_hbm, trows_hbm, irows_hbm):
        def tbody(i_vmem, o_vmem):
            pltpu.sync_copy(ttab_hbm.at[i_vmem.at[0]], o_vmem)

        pltpu.emit_pipeline(
            tbody,
            grid=(n_t // _GW,),
            in_specs=[pl.BlockSpec((1, _GW), lambda i: (0, i))],
            out_specs=[pl.BlockSpec((_GW, _DT), lambda i: (i, 0))],
            core_axis_name=("c", "s"),
            dimension_semantics=(pltpu.PARALLEL,),
        )(tidx_hbm, trows_hbm)

        def ibody(i_vmem, o_vmem):
            pltpu.sync_copy(itab_hbm.at[i_vmem.at[0]], o_vmem)

        pltpu.emit_pipeline(
            ibody,
            grid=(n_i // _GW,),
            in_specs=[pl.BlockSpec((1, _GW), lambda i: (0, i))],
            out_specs=[pl.BlockSpec((_GW, _DI), lambda i: (i, 0))],
            core_axis_name=("c", "s"),
            dimension_semantics=(pltpu.PARALLEL,),
        )(iidx_hbm, irows_hbm)

    return k(title_table, movie_id_table, tidx2d, iidx2d)


def _tc_body(tidx_ref, gidx_ref, trows_ref, irows_ref, t0_ref, gtab_ref,
             w1_ref, b1_ref, w2_ref, b2_ref, out_ref):
    trows = trows_ref[...]
    tsum = jnp.sum(trows, axis=1)
    tmask = (tidx_ref[...] != 0).astype(jnp.float32)
    tcnt = jnp.sum(tmask, axis=1)
    n0 = _LT - tcnt
    pooled_t = (tsum - n0[:, None] * t0_ref[...]) / jnp.maximum(tcnt, 1.0)[:, None]

    g = gidx_ref[...]
    vg_iota = lax.broadcasted_iota(jnp.int32, (1, 1, _VG), 2)
    onehot = ((g[:, :, None] == vg_iota) & (g[:, :, None] != 0)).astype(jnp.float32)
    counts = jnp.sum(onehot, axis=1)
    gcnt = jnp.sum((g != 0).astype(jnp.float32), axis=1)
    gsum = jnp.dot(counts, gtab_ref[...], preferred_element_type=jnp.float32)
    pooled_g = gsum / jnp.maximum(gcnt, 1.0)[:, None]

    w1 = w1_ref[...]
    h = (jnp.dot(pooled_t, w1[0:_DT], preferred_element_type=jnp.float32)
         + jnp.dot(irows_ref[...], w1[_DT:_DT + _DI], preferred_element_type=jnp.float32)
         + jnp.dot(pooled_g, w1[_DT + _DI:], preferred_element_type=jnp.float32)
         + b1_ref[...])
    h = jnp.maximum(h, 0.0)
    out = jnp.dot(h, w2_ref[...], preferred_element_type=jnp.float32) + b2_ref[...]
    out = jnp.maximum(out, 0.0)
    sq = jnp.sum(out * out, axis=-1, keepdims=True)
    out_ref[...] = out * lax.rsqrt(jnp.maximum(sq, 1e-12))


def kernel(movie_id, movie_title, movie_genres, movie_id_table, title_table,
           genre_table, W1, b1, W2, b2):
    tidx = movie_title.astype(jnp.int32)
    iidx = movie_id.astype(jnp.int32)
    gidx = movie_genres.astype(jnp.int32)

    trows, irows = _sc_gather(
        title_table, movie_id_table,
        tidx.reshape(1, _B * _LT), iidx.reshape(1, _B),
    )
    trows = trows.reshape(_B, _LT, _DT)

    grid = (_B // _BB,)
    out = pl.pallas_call(
        _tc_body,
        grid=grid,
        in_specs=[
            pl.BlockSpec((_BB, _LT), lambda i: (i, 0)),
            pl.BlockSpec((_BB, _LG), lambda i: (i, 0)),
            pl.BlockSpec((_BB, _LT, _DT), lambda i: (i, 0, 0)),
            pl.BlockSpec((_BB, _DI), lambda i: (i, 0)),
            pl.BlockSpec((1, _DT), lambda i: (0, 0)),
            pl.BlockSpec((_VG, _DG), lambda i: (0, 0)),
            pl.BlockSpec((_DT + _DI + _DG, 64), lambda i: (0, 0)),
            pl.BlockSpec((1, 64), lambda i: (0, 0)),
            pl.BlockSpec((64, 32), lambda i: (0, 0)),
            pl.BlockSpec((1, 32), lambda i: (0, 0)),
        ],
        out_specs=pl.BlockSpec((_BB, 32), lambda i: (i, 0)),
        out_shape=jax.ShapeDtypeStruct((_B, 32), jnp.float32),
    )(tidx, gidx, trows, irows, title_table[0:1], genre_table,
      W1, b1.reshape(1, 64), W2, b2.reshape(1, 32))
    return out


# speedup vs baseline: 1.4361x; 1.0192x over previous
import functools

import jax
import jax.numpy as jnp
from jax import lax
from jax.experimental import pallas as pl
from jax.experimental.pallas import tpu as pltpu
from jax.experimental.pallas import tpu_sc as plsc

_B = 4096
_LT = 20
_LG = 8
_DT = 64
_DI = 64
_DG = 32
_VG = 32
_GW = 128
_BB = 512


def _sc_gather(t2, i2, tidx2d, iidx2d):
    n_t = tidx2d.shape[1]
    n_i = iidx2d.shape[1]
    mesh = plsc.VectorSubcoreMesh(core_axis_name="c", subcore_axis_name="s")

    @functools.partial(
        pl.kernel,
        out_type=(
            jax.ShapeDtypeStruct((n_t, 128), jnp.float32),
            jax.ShapeDtypeStruct((n_i, 128), jnp.float32),
        ),
        mesh=mesh,
    )
    def k(ttab_hbm, itab_hbm, tidx_hbm, iidx_hbm, trows_hbm, irows_hbm):
        def tbody(i_vmem, o_vmem):
            pltpu.sync_copy(ttab_hbm.at[i_vmem.at[0]], o_vmem)

        pltpu.emit_pipeline(
            tbody,
            grid=(n_t // _GW,),
            in_specs=[pl.BlockSpec((1, _GW), lambda i: (0, i))],
            out_specs=[pl.BlockSpec((_GW, 128), lambda i: (i, 0))],
            core_axis_name=("c", "s"),
            dimension_semantics=(pltpu.PARALLEL,),
        )(tidx_hbm, trows_hbm)

        def ibody(i_vmem, o_vmem):
            pltpu.sync_copy(itab_hbm.at[i_vmem.at[0]], o_vmem)

        pltpu.emit_pipeline(
            ibody,
            grid=(n_i // _GW,),
            in_specs=[pl.BlockSpec((1, _GW), lambda i: (0, i))],
            out_specs=[pl.BlockSpec((_GW, 128), lambda i: (i, 0))],
            core_axis_name=("c", "s"),
            dimension_semantics=(pltpu.PARALLEL,),
        )(iidx_hbm, irows_hbm)

    return k(t2, i2, tidx2d, iidx2d)


def _tc_body(tidx_ref, pid_ref, gidx_ref, trows_ref, irows_ref, t0_ref,
             gtab_ref, w1_ref, b1_ref, w2_ref, b2_ref, out_ref):
    tidx = tidx_ref[...]
    tsum = jnp.zeros((tidx.shape[0], _DT), jnp.float32)
    for l in range(_LT):
        row = trows_ref[l]
        podd = (tidx[:, l:l + 1] & 1) == 1
        tsum = tsum + jnp.where(podd, row[:, _DT:], row[:, :_DT])
    tcnt = jnp.sum((tidx != 0).astype(jnp.float32), axis=1)
    n0 = _LT - tcnt
    t0 = t0_ref[0:1, :_DT]
    pooled_t = (tsum - n0[:, None] * t0) / jnp.maximum(tcnt, 1.0)[:, None]

    irows = irows_ref[...]
    iodd = (pid_ref[...][:, 0:1] & 1) == 1
    id_emb = jnp.where(iodd, irows[:, _DT:], irows[:, :_DT])

    g = gidx_ref[...]
    vg_iota = lax.broadcasted_iota(jnp.int32, (1, 1, _VG), 2)
    onehot = ((g[:, :, None] == vg_iota) & (g[:, :, None] != 0)).astype(jnp.float32)
    counts = jnp.sum(onehot, axis=1)
    gcnt = jnp.sum((g != 0).astype(jnp.float32), axis=1)
    gsum = jnp.dot(counts, gtab_ref[...], preferred_element_type=jnp.float32)
    pooled_g = gsum / jnp.maximum(gcnt, 1.0)[:, None]

    w1 = w1_ref[...]
    h = (jnp.dot(pooled_t, w1[0:_DT], preferred_element_type=jnp.float32)
         + jnp.dot(id_emb, w1[_DT:_DT + _DI], preferred_element_type=jnp.float32)
         + jnp.dot(pooled_g, w1[_DT + _DI:], preferred_element_type=jnp.float32)
         + b1_ref[...])
    h = jnp.maximum(h, 0.0)
    out = jnp.dot(h, w2_ref[...], preferred_element_type=jnp.float32) + b2_ref[...]
    out = jnp.maximum(out, 0.0)
    sq = jnp.sum(out * out, axis=-1, keepdims=True)
    out_ref[...] = out * lax.rsqrt(jnp.maximum(sq, 1e-12))


def kernel(movie_id, movie_title, movie_genres, movie_id_table, title_table,
           genre_table, W1, b1, W2, b2):
    t2 = title_table.reshape(50000, 128)
    i2 = movie_id_table.reshape(50000, 128)
    tidx = movie_title.astype(jnp.int32)
    tidxT = tidx.T
    iidx = movie_id.astype(jnp.int32)
    gidx = movie_genres.astype(jnp.int32)

    trows, irows = _sc_gather(
        t2, i2,
        (tidxT >> 1).reshape(1, _B * _LT), (iidx >> 1).reshape(1, _B),
    )
    trows3 = trows.reshape(_LT, _B, 128)

    grid = (_B // _BB,)
    out = pl.pallas_call(
        _tc_body,
        grid=grid,
        in_specs=[
            pl.BlockSpec((_BB, _LT), lambda i: (i, 0)),
            pl.BlockSpec((_BB, 1), lambda i: (i, 0)),
            pl.BlockSpec((_BB, _LG), lambda i: (i, 0)),
            pl.BlockSpec((_LT, _BB, 128), lambda i: (0, i, 0)),
            pl.BlockSpec((_BB, 128), lambda i: (i, 0)),
            pl.BlockSpec((1, 128), lambda i: (0, 0)),
            pl.BlockSpec((_VG, _DG), lambda i: (0, 0)),
            pl.BlockSpec((_DT + _DI + _DG, 64), lambda i: (0, 0)),
            pl.BlockSpec((1, 64), lambda i: (0, 0)),
            pl.BlockSpec((64, 32), lambda i: (0, 0)),
            pl.BlockSpec((1, 32), lambda i: (0, 0)),
        ],
        out_specs=pl.BlockSpec((_BB, 32), lambda i: (i, 0)),
        out_shape=jax.ShapeDtypeStruct((_B, 32), jnp.float32),
        compiler_params=pltpu.CompilerParams(
            dimension_semantics=("arbitrary",)),
    )(tidx, iidx.reshape(_B, 1), gidx, trows3, irows, t2[0:1], genre_table,
      W1, b1.reshape(1, 64), W2, b2.reshape(1, 32))
    return out
